# baseline (speedup 1.0000x reference)
import functools

import jax
import jax.numpy as jnp
from jax import lax
from jax.experimental import pallas as pl
from jax.experimental.pallas import tpu as pltpu
from jax.experimental.pallas import tpu_sc as plsc

B = 16384
DIM = 32
NC = 2
NS = 16
NW = NC * NS
BPW = B // NW
L = 16
NGRP = BPW // L
CHUNK = 128
NCHUNK = BPW // CHUNK


def _body(pairs_hbm, emb_hbm, ctx_hbm, out_hbm,
          pv, cidx, tidx, erows, crows, outv, sem):
    wid = lax.axis_index("s") * NC + lax.axis_index("c")
    base = wid * BPW

    pltpu.sync_copy(pairs_hbm.at[pl.ds(2 * base, 2 * BPW)], pv)

    iota = lax.iota(jnp.int32, L)
    for g in range(NGRP):
        flat = 2 * (g * L + iota)
        cid = plsc.load_gather(pv, [flat])
        tid = plsc.load_gather(pv, [flat + 1])
        j, c0 = (g * L) // CHUNK, (g * L) % CHUNK
        cidx[j, pl.ds(c0, L)] = cid
        tidx[j, pl.ds(c0, L)] = tid

    copies = []
    for j in range(NCHUNK):
        dst_e = erows.at[pl.ds(j * CHUNK, CHUNK), :]
        dst_c = crows.at[pl.ds(j * CHUNK, CHUNK), :]
        copies.append(pltpu.async_copy(emb_hbm.at[cidx.at[j]], dst_e, sem))
        copies.append(pltpu.async_copy(ctx_hbm.at[tidx.at[j]], dst_c, sem))
    for cp in copies:
        cp.wait()

    dcols = [jnp.full((L,), d, jnp.int32) for d in range(DIM)]

    def group(g, _):
        rows = g * L + iota
        acc = jnp.zeros((L,), jnp.float32)
        for d in range(DIM):
            a = plsc.load_gather(erows, [rows, dcols[d]])
            b = plsc.load_gather(crows, [rows, dcols[d]])
            acc = acc + a * b
        y = 1.0 / (1.0 + jnp.exp(-acc))
        plsc.store_scatter(outv, [rows], y)
        return 0

    lax.fori_loop(0, NGRP, group, 0)

    pltpu.sync_copy(outv, out_hbm.at[pl.ds(base, BPW)])


@functools.partial(jax.jit, donate_argnums=())
def _skipgram(pairs, emb_w, ctx_w):
    mesh = plsc.VectorSubcoreMesh(core_axis_name="c", subcore_axis_name="s")
    k = pl.kernel(
        _body,
        out_type=jax.ShapeDtypeStruct((B,), jnp.float32),
        mesh=mesh,
        compiler_params=pltpu.CompilerParams(
            needs_layout_passes=False, use_tc_tiling_on_sc=False),
        scratch_types=[
            pltpu.VMEM((2 * BPW,), jnp.int32),
            pltpu.VMEM((NCHUNK, CHUNK), jnp.int32),
            pltpu.VMEM((NCHUNK, CHUNK), jnp.int32),
            pltpu.VMEM((BPW, DIM), jnp.float32),
            pltpu.VMEM((BPW, DIM), jnp.float32),
            pltpu.VMEM((BPW,), jnp.float32),
            pltpu.SemaphoreType.DMA,
        ],
    )
    return k(pairs, emb_w, ctx_w)


def kernel(pairs, emb_w, ctx_w):
    return _skipgram(pairs.astype(jnp.int32).reshape(-1), emb_w, ctx_w)
